# baseline (speedup 1.0000x reference)
import jax
import jax.numpy as jnp
from jax import lax
from jax.experimental import pallas as pl
from jax.experimental.pallas import tpu as pltpu
from jax.experimental.pallas import tpu_sc as plsc

B, L, D = 4096, 200, 64
NC, NS = 2, 16
NW = NC * NS
ROWS_PW = B // NW
HALF = L // 2


def _emb_body(idx_hbm, table_hbm, pos_hbm, out_hbm, idx_v, pos_v, rows_v,
              gsem):
    wid = lax.axis_index("s") * NC + lax.axis_index("c")
    pltpu.sync_copy(pos_hbm, pos_v)
    pltpu.sync_copy(idx_hbm.at[pl.ds(wid * 2 * ROWS_PW, 2 * ROWS_PW)], idx_v)

    def row_body(r, carry):
        row = wid * ROWS_PW + r
        c0 = pltpu.async_copy(table_hbm.at[idx_v.at[2 * r]],
                              rows_v.at[pl.ds(0, HALF)], gsem)
        c1 = pltpu.async_copy(table_hbm.at[idx_v.at[2 * r + 1]],
                              rows_v.at[pl.ds(HALF, HALF)], gsem)
        c0.wait()
        c1.wait()

        def add_body(i, c):
            for q in range(D // 16):
                sl = pl.ds(q * 16, 16)
                plsc.addupdate(rows_v.at[i, sl], pos_v[i, sl])
            return c

        lax.fori_loop(0, L, add_body, 0, unroll=2)
        pltpu.sync_copy(rows_v, out_hbm.at[row])
        return carry

    lax.fori_loop(0, ROWS_PW, row_body, 0)


def kernel(embedding_idx, token_table, pos_table):
    idx2 = embedding_idx.astype(jnp.int32).reshape(2 * B, HALF)
    mesh = plsc.VectorSubcoreMesh(core_axis_name="c", subcore_axis_name="s")
    k = pl.kernel(
        _emb_body,
        out_type=jax.ShapeDtypeStruct((B, L, D), jnp.float32),
        mesh=mesh,
        scratch_types=[
            pltpu.VMEM((2 * ROWS_PW, HALF), jnp.int32),
            pltpu.VMEM((L, D), jnp.float32),
            pltpu.VMEM((L, D), jnp.float32),
            pltpu.SemaphoreType.DMA,
        ],
        compiler_params=pltpu.CompilerParams(use_tc_tiling_on_sc=False),
    )
    return k(idx2, token_table, pos_table)


# speedup vs baseline: 1.1640x; 1.1640x over previous
import jax
import jax.numpy as jnp
from jax import lax
from jax.experimental import pallas as pl
from jax.experimental.pallas import tpu as pltpu
from jax.experimental.pallas import tpu_sc as plsc

B, L, D = 4096, 200, 64
NC, NS = 2, 16
NW = NC * NS
ROWS_PW = B // NW
HALF = L // 2
NBUF = 4
AHEAD = 2


def _emb_body(idx_hbm, table_hbm, pos_hbm, out_hbm, idx_v, pos_v, rows_v,
              gsem, osem):
    wid = lax.axis_index("s") * NC + lax.axis_index("c")
    row0 = wid * ROWS_PW
    pltpu.sync_copy(pos_hbm, pos_v)
    pltpu.sync_copy(idx_hbm.at[pl.ds(wid * 2 * ROWS_PW, 2 * ROWS_PW)], idx_v)

    def fire_gathers(r, b):
        pltpu.async_copy(table_hbm.at[idx_v.at[2 * r]],
                         rows_v.at[b, pl.ds(0, HALF)], gsem.at[b])
        pltpu.async_copy(table_hbm.at[idx_v.at[2 * r + 1]],
                         rows_v.at[b, pl.ds(HALF, HALF)], gsem.at[b])

    def wait_gathers(r, b):
        pltpu.make_async_copy(table_hbm.at[idx_v.at[2 * r]],
                              rows_v.at[b, pl.ds(0, HALF)], gsem.at[b]).wait()
        pltpu.make_async_copy(table_hbm.at[idx_v.at[2 * r + 1]],
                              rows_v.at[b, pl.ds(HALF, HALF)],
                              gsem.at[b]).wait()

    def wait_out(r, b):
        pltpu.make_async_copy(rows_v.at[b], out_hbm.at[row0 + r],
                              osem.at[b]).wait()

    for r in range(AHEAD):
        fire_gathers(r, r % NBUF)

    def row_body(r, carry):
        rn = r + AHEAD

        @pl.when(rn < ROWS_PW)
        def _():
            bn = lax.rem(rn, NBUF)

            @pl.when(rn >= NBUF)
            def _():
                wait_out(rn - NBUF, bn)

            fire_gathers(rn, bn)

        b = lax.rem(r, NBUF)
        wait_gathers(r, b)

        def add_body(i):
            for q in range(D // 16):
                sl = pl.ds(q * 16, 16)
                plsc.addupdate(rows_v.at[b, i, sl], pos_v[i, sl])

        plsc.parallel_loop(0, L, 1, unroll=4)(add_body)
        pltpu.async_copy(rows_v.at[b], out_hbm.at[row0 + r], osem.at[b])
        return carry

    lax.fori_loop(0, ROWS_PW, row_body, 0)

    for k in range(NBUF):
        r = ROWS_PW - NBUF + k
        wait_out(r, r % NBUF)


def kernel(embedding_idx, token_table, pos_table):
    idx2 = embedding_idx.astype(jnp.int32).reshape(2 * B, HALF)
    mesh = plsc.VectorSubcoreMesh(core_axis_name="c", subcore_axis_name="s")
    k = pl.kernel(
        _emb_body,
        out_type=jax.ShapeDtypeStruct((B, L, D), jnp.float32),
        mesh=mesh,
        scratch_types=[
            pltpu.VMEM((2 * ROWS_PW, HALF), jnp.int32),
            pltpu.VMEM((L, D), jnp.float32),
            pltpu.VMEM((NBUF, L, D), jnp.float32),
            pltpu.SemaphoreType.DMA((NBUF,)),
            pltpu.SemaphoreType.DMA((NBUF,)),
        ],
        compiler_params=pltpu.CompilerParams(use_tc_tiling_on_sc=False),
    )
    return k(idx2, token_table, pos_table)


# speedup vs baseline: 1.1977x; 1.0289x over previous
import jax
import jax.numpy as jnp
from jax import lax
from jax.experimental import pallas as pl
from jax.experimental.pallas import tpu as pltpu
from jax.experimental.pallas import tpu_sc as plsc

B, L, D = 4096, 200, 64
NC, NS = 2, 16
NW = NC * NS
BT = B // NW
NBUF = 4
AHEAD = 2


def _emb_body(idx_hbm, table_hbm, pos_hbm, out_hbm, idx_v, pos_v, rows_v,
              gsem, osem):
    wid = lax.axis_index("s") * NC + lax.axis_index("c")
    b0 = wid * BT
    pltpu.sync_copy(pos_hbm, pos_v)
    pltpu.sync_copy(idx_hbm.at[:, pl.ds(b0, BT)], idx_v)

    def fire_gather(l, b):
        pltpu.async_copy(table_hbm.at[idx_v.at[l]], rows_v.at[b], gsem.at[b])

    def wait_gather(l, b):
        pltpu.make_async_copy(table_hbm.at[idx_v.at[l]], rows_v.at[b],
                              gsem.at[b]).wait()

    def wait_out(l, b):
        pltpu.make_async_copy(rows_v.at[b], out_hbm.at[l, pl.ds(b0, BT)],
                              osem.at[b]).wait()

    for l in range(AHEAD):
        fire_gather(l, l % NBUF)

    def pos_body(l, carry):
        ln = l + AHEAD

        @pl.when(ln < L)
        def _():
            bn = lax.rem(ln, NBUF)

            @pl.when(ln >= NBUF)
            def _():
                wait_out(ln - NBUF, bn)

            fire_gather(ln, bn)

        b = lax.rem(l, NBUF)
        wait_gather(l, b)

        pv = [pos_v[l, pl.ds(q * 16, 16)] for q in range(D // 16)]

        def add_body(i):
            for q in range(D // 16):
                plsc.addupdate(rows_v.at[b, i, pl.ds(q * 16, 16)], pv[q])

        plsc.parallel_loop(0, BT, 1, unroll=4)(add_body)
        pltpu.async_copy(rows_v.at[b], out_hbm.at[l, pl.ds(b0, BT)],
                         osem.at[b])
        return carry

    lax.fori_loop(0, L, pos_body, 0)

    for k in range(NBUF):
        l = L - NBUF + k
        wait_out(l, l % NBUF)


def kernel(embedding_idx, token_table, pos_table):
    idx_t = embedding_idx.astype(jnp.int32).T
    mesh = plsc.VectorSubcoreMesh(core_axis_name="c", subcore_axis_name="s")
    k = pl.kernel(
        _emb_body,
        out_type=jax.ShapeDtypeStruct((L, B, D), jnp.float32),
        mesh=mesh,
        scratch_types=[
            pltpu.VMEM((L, BT), jnp.int32),
            pltpu.VMEM((L, D), jnp.float32),
            pltpu.VMEM((NBUF, BT, D), jnp.float32),
            pltpu.SemaphoreType.DMA((NBUF,)),
            pltpu.SemaphoreType.DMA((NBUF,)),
        ],
        compiler_params=pltpu.CompilerParams(use_tc_tiling_on_sc=False),
    )
    out_t = k(idx_t, token_table, pos_table)
    return out_t.transpose(1, 0, 2)
